# Initial kernel scaffold; baseline (speedup 1.0000x reference)
import jax
import jax.numpy as jnp
from jax.experimental import pallas as pl


def kernel(x, edge_index, W1, b1, W2, b2):
    raise NotImplementedError("write your pallas kernel here")


# speedup vs baseline: 8.2456x; 8.2456x over previous
import functools

import jax
import jax.numpy as jnp
from jax import lax
from jax.experimental import pallas as pl
from jax.experimental.pallas import tpu as pltpu
from jax.experimental.pallas import tpu_sc as plsc

N = 10000
T = 4
D = 128
E = 160000
K = 4
NSC = 2
NTL = 16
BATCH = 128
ROWS_PER_TILE = 624
CHUNK_ROWS = 208
TAIL_ROWS = N - NTL * ROWS_PER_TILE

_mesh = plsc.VectorSubcoreMesh(core_axis_name="c", subcore_axis_name="s")


def _i32(v):
    return jnp.asarray(v, jnp.int32)


@functools.partial(
    pl.kernel,
    out_type=jax.ShapeDtypeStruct((NSC * N, 16), jnp.float32),
    mesh=_mesh,
    scratch_types=[
        pltpu.VMEM_SHARED((N, 16), jnp.float32),
        pltpu.VMEM((BATCH, 16), jnp.float32),
        pltpu.VMEM((1, BATCH), jnp.int32),
        pltpu.VMEM((CHUNK_ROWS, 16), jnp.float32),
    ],
)
def _sc_degree(dst_hbm, out_hbm, acc, ones_v, dst_v, stage_v):
    c = lax.axis_index("c")
    s = lax.axis_index("s")
    half = E // NSC
    nb = half // BATCH
    rem = nb % NTL
    trips = nb // NTL + jnp.where(s < rem, 1, 0).astype(jnp.int32)

    for i in range(BATCH):
        ones_v[_i32(i), :] = jnp.full((16,), 1.0, jnp.float32)
    for i in range(CHUNK_ROWS):
        stage_v[_i32(i), :] = jnp.zeros((16,), jnp.float32)

    for q in range(ROWS_PER_TILE // CHUNK_ROWS):
        r = s * ROWS_PER_TILE + q * CHUNK_ROWS
        pltpu.sync_copy(stage_v, acc.at[pl.ds(r, CHUNK_ROWS)])

    @pl.when(s == NTL - 1)
    def _():
        pltpu.sync_copy(stage_v.at[pl.ds(0, TAIL_ROWS)],
                        acc.at[pl.ds(_i32(NTL * ROWS_PER_TILE), TAIL_ROWS)])
    plsc.subcore_barrier()

    def body(j, carry):
        off = (c * half) + (s + NTL * j) * BATCH
        pltpu.sync_copy(dst_hbm.at[pl.ds(off, BATCH)], dst_v.at[_i32(0)])
        pltpu.sync_copy(ones_v, acc.at[dst_v.at[_i32(0)]], add=True)
        return carry

    lax.fori_loop(_i32(0), trips, body, _i32(0))
    plsc.subcore_barrier()

    for q in range(ROWS_PER_TILE // CHUNK_ROWS):
        r = s * ROWS_PER_TILE + q * CHUNK_ROWS
        pltpu.sync_copy(acc.at[pl.ds(r, CHUNK_ROWS)], stage_v)
        pltpu.sync_copy(stage_v, out_hbm.at[pl.ds(c * N + r, CHUNK_ROWS)])

    @pl.when(s == NTL - 1)
    def _():
        t0 = _i32(NTL * ROWS_PER_TILE)
        pltpu.sync_copy(acc.at[pl.ds(t0, TAIL_ROWS)],
                        stage_v.at[pl.ds(0, TAIL_ROWS)])
        pltpu.sync_copy(stage_v.at[pl.ds(0, TAIL_ROWS)],
                        out_hbm.at[pl.ds(c * N + t0, TAIL_ROWS)])


@functools.partial(
    pl.kernel,
    out_type=jax.ShapeDtypeStruct((K * N, D), jnp.float32),
    mesh=_mesh,
    scratch_types=[
        pltpu.VMEM_SHARED((N, D), jnp.float32),
        pltpu.VMEM((BATCH,), jnp.int32),
        pltpu.VMEM((1, BATCH), jnp.int32),
        pltpu.VMEM((BATCH, D), jnp.float32),
        pltpu.VMEM((CHUNK_ROWS, D), jnp.float32),
        pltpu.SemaphoreType.DMA,
    ],
)
def _sc_layer(h_hbm, src_hbm, dst_hbm, out_hbm, acc, src_v, dst_v, rows_v,
              stage_v, sem):
    c = lax.axis_index("c")
    s = lax.axis_index("s")
    nb = E // BATCH
    rem = nb % NTL
    trips = nb // NTL + jnp.where(s < rem, 1, 0).astype(jnp.int32)

    for ki in range(K // NSC):
        k = c + NSC * ki
        kn = k * N

        for q in range(ROWS_PER_TILE // CHUNK_ROWS):
            r = s * ROWS_PER_TILE + q * CHUNK_ROWS
            pltpu.sync_copy(h_hbm.at[pl.ds(kn + r, CHUNK_ROWS)], stage_v)
            pltpu.sync_copy(stage_v, acc.at[pl.ds(r, CHUNK_ROWS)])

        @pl.when(s == NTL - 1)
        def _():
            t0 = _i32(NTL * ROWS_PER_TILE)
            pltpu.sync_copy(h_hbm.at[pl.ds(kn + t0, TAIL_ROWS)],
                            stage_v.at[pl.ds(0, TAIL_ROWS)])
            pltpu.sync_copy(stage_v.at[pl.ds(0, TAIL_ROWS)],
                            acc.at[pl.ds(t0, TAIL_ROWS)])
        plsc.subcore_barrier()

        def body(j, carry):
            off = (s + NTL * j) * BATCH
            pltpu.sync_copy(src_hbm.at[pl.ds(off, BATCH)], src_v)
            pltpu.sync_copy(dst_hbm.at[pl.ds(off, BATCH)], dst_v.at[_i32(0)])
            for i in range(BATCH // 16):
                sl = pl.ds(i * 16, 16)
                src_v[sl] = src_v[sl] + kn
            pltpu.async_copy(h_hbm.at[src_v], rows_v, sem).wait()
            pltpu.sync_copy(rows_v, acc.at[dst_v.at[_i32(0)]], add=True)
            return carry

        lax.fori_loop(_i32(0), trips, body, _i32(0))
        plsc.subcore_barrier()

        for q in range(ROWS_PER_TILE // CHUNK_ROWS):
            r = s * ROWS_PER_TILE + q * CHUNK_ROWS
            pltpu.sync_copy(acc.at[pl.ds(r, CHUNK_ROWS)], stage_v)
            pltpu.sync_copy(stage_v, out_hbm.at[pl.ds(kn + r, CHUNK_ROWS)])

        @pl.when(s == NTL - 1)
        def _():
            t0 = _i32(NTL * ROWS_PER_TILE)
            pltpu.sync_copy(acc.at[pl.ds(t0, TAIL_ROWS)],
                            stage_v.at[pl.ds(0, TAIL_ROWS)])
            pltpu.sync_copy(stage_v.at[pl.ds(0, TAIL_ROWS)],
                            out_hbm.at[pl.ds(kn + t0, TAIL_ROWS)])
        plsc.subcore_barrier()


NB = 2000
_NBLK = N // NB


def _dis(p0, p1):
    deg = p0[:, :1] + p1[:, :1] + 1.0
    return lax.rsqrt(deg)


def _pre_body(x_ref, w_ref, p0_ref, p1_ref, o_ref):
    d = _dis(p0_ref[...], p1_ref[...])
    o_ref[...] = jnp.dot(x_ref[...], w_ref[...],
                         preferred_element_type=jnp.float32) * d


def _mid_body(a_ref, w_ref, b_ref, p0_ref, p1_ref, o_ref):
    d = _dis(p0_ref[...], p1_ref[...])
    y = jnp.maximum(a_ref[...] * d + b_ref[...], 0.0)
    o_ref[...] = jnp.dot(y, w_ref[...],
                         preferred_element_type=jnp.float32) * d


def _post_body(a_ref, b_ref, p0_ref, p1_ref, o_ref):
    d = _dis(p0_ref[...], p1_ref[...])
    o_ref[...] = jnp.maximum(a_ref[...] * d + b_ref[...], 0.0)


_Z = lambda: jnp.asarray(0, jnp.int32)
_row_spec = pl.BlockSpec((NB, D), lambda g: (g, _Z()))
_w_spec = pl.BlockSpec((D, D), lambda g: (_Z(), _Z()))
_b_spec = pl.BlockSpec((1, D), lambda g: (_Z(), _Z()))
_p0_spec = pl.BlockSpec((NB, 16), lambda g: (g % _i32(_NBLK), _Z()))
_p1_spec = pl.BlockSpec((NB, 16), lambda g: (_i32(_NBLK) + g % _i32(_NBLK), _Z()))
_out_rows = jax.ShapeDtypeStruct((K * N, D), jnp.float32)
_GRID = (K * N // NB,)


def _tc_pre(x_rows, w1, degp):
    return pl.pallas_call(
        _pre_body, grid=_GRID,
        in_specs=[_row_spec, _w_spec, _p0_spec, _p1_spec],
        out_specs=_row_spec, out_shape=_out_rows,
    )(x_rows, w1, degp, degp)


def _tc_mid(a_rows, w2, b1, degp):
    return pl.pallas_call(
        _mid_body, grid=_GRID,
        in_specs=[_row_spec, _w_spec, _b_spec, _p0_spec, _p1_spec],
        out_specs=_row_spec, out_shape=_out_rows,
    )(a_rows, w2, b1, degp, degp)


def _tc_post(a_rows, b2, degp):
    return pl.pallas_call(
        _post_body, grid=_GRID,
        in_specs=[_row_spec, _b_spec, _p0_spec, _p1_spec],
        out_specs=_row_spec, out_shape=_out_rows,
    )(a_rows, b2, degp, degp)


def kernel(x, edge_index, W1, b1, W2, b2):
    src = edge_index[0].astype(jnp.int32)
    dst = edge_index[1].astype(jnp.int32)
    x_rows = (x.reshape(2, N // 2, 2, 2, D)
              .transpose(2, 0, 1, 3, 4).reshape(K * N, D))

    degp = _sc_degree(dst)
    h1 = _tc_pre(x_rows, W1, degp)
    a1 = _sc_layer(h1, src, dst)
    h2 = _tc_mid(a1, W2, b1.reshape(1, D), degp)
    a2 = _sc_layer(h2, src, dst)
    o = _tc_post(a2, b2.reshape(1, D), degp)
    return (o.reshape(2, 2, N // 2, 2, D)
            .transpose(1, 2, 0, 3, 4).reshape(N, T, D))
